# baseline (speedup 1.0000x reference)
import functools

import jax
import jax.numpy as jnp
from jax import lax
from jax.experimental import pallas as pl
from jax.experimental.pallas import tpu as pltpu
from jax.experimental.pallas import tpu_sc as plsc

B = 4096
V = 100000
NC = 2
NS = 16
RPW = B // NS
L = 16
NG = RPW // L
QR = RPW // 4


def _sc_loss(inp_t, tgt_flat):
    mesh = plsc.VectorSubcoreMesh(core_axis_name="c", subcore_axis_name="s")

    @functools.partial(
        pl.kernel,
        out_type=jax.ShapeDtypeStruct((L,), jnp.float32),
        mesh=mesh,
        scratch_types=[
            pltpu.VMEM((RPW,), jnp.int32),
            pltpu.VMEM((RPW, 128), jnp.float32),
            pltpu.VMEM((L,), jnp.float32),
            pltpu.VMEM((NS * L,), jnp.float32),
            pltpu.VMEM_SHARED((NS * L,), jnp.float32),
            pltpu.SemaphoreType.DMA,
            pltpu.SemaphoreType.DMA,
            pltpu.SemaphoreType.DMA,
            pltpu.SemaphoreType.DMA,
            pltpu.SemaphoreType.DMA,
            pltpu.SemaphoreType.DMA,
        ],
    )
    def k(inp_hbm, tgt_hbm, out_hbm, tgt_v, val_v, stage_v, comb_v, shared,
          st0, st1, g0, g1, g2, g3):
        cid = lax.axis_index("c")
        sid = lax.axis_index("s")
        base = sid * RPW
        half = RPW // 2
        tcs = [
            pltpu.make_async_copy(
                tgt_hbm.at[pl.ds(base + h * half, half)],
                tgt_v.at[pl.ds(h * half, half)],
                s,
            )
            for h, s in enumerate([st0, st1])
        ]
        tcs[0].start()
        tcs[1].start()
        gsem = [g0, g1, g2, g3]
        cps = []
        for q in range(4):
            blk = pl.ds(pl.multiple_of(base + (q // 2) * 128, 128), 128)
            cps.append(
                pltpu.make_async_copy(
                    inp_hbm.at[tgt_v.at[pl.ds(q * QR, QR)], blk],
                    val_v.at[pl.ds(q * QR, QR)],
                    gsem[q],
                )
            )
        tcs[0].wait()
        cps[0].start()
        cps[1].start()
        tcs[1].wait()
        cps[2].start()
        cps[3].start()
        lanes = lax.iota(jnp.int32, L)

        def grp_body(g, acc):
            tch = tgt_v[pl.ds(g * L, L)]
            grp = jnp.zeros((L,), jnp.float32)
            for j in range(L):
                chunk = val_v[g * L + j, pl.ds((g % 8) * L, L)]
                grp = grp + jnp.where(lanes == j, chunk, 0.0)
            return acc + jnp.where(tch > 0, grp, 0.0)

        acc = jnp.zeros((L,), jnp.float32)
        for q in range(4):
            cps[q].wait()
            acc = plsc.parallel_loop(q * 4, q * 4 + 4, carry=acc)(grp_body)
        stage_v[...] = acc
        pltpu.sync_copy(stage_v, shared.at[pl.ds(pl.multiple_of(sid * L, L), L)])
        plsc.subcore_barrier()

        @pl.when(jnp.logical_and(cid == 0, sid == 0))
        def _():
            pltpu.sync_copy(shared, comb_v)
            tv = comb_v[pl.ds(0, L)]
            for s in range(1, NS):
                tv = tv + comb_v[pl.ds(s * L, L)]
            tot = tv[0]
            for i in range(1, L):
                tot = tot + tv[i]
            stage_v[...] = jnp.where(lanes == 0, -tot, 0.0)
            pltpu.sync_copy(stage_v, out_hbm)

    return k(inp_t, tgt_flat)


def kernel(input, target):
    tgt = target.reshape(-1).astype(jnp.int32)
    out = _sc_loss(input.T, tgt)
    return out[0]


# speedup vs baseline: 1.0455x; 1.0455x over previous
import functools

import jax
import jax.numpy as jnp
from jax import lax
from jax.experimental import pallas as pl
from jax.experimental.pallas import tpu as pltpu
from jax.experimental.pallas import tpu_sc as plsc

B = 4096
V = 100000
NC = 2
NS = 16
NW = NC * NS
RPW = B // NW
L = 16
NG = RPW // L


def _sc_gather_partials(inp_t, tgt_flat):
    mesh = plsc.VectorSubcoreMesh(core_axis_name="c", subcore_axis_name="s")

    @functools.partial(
        pl.kernel,
        out_type=jax.ShapeDtypeStruct((NW * L,), jnp.float32),
        mesh=mesh,
        scratch_types=[
            pltpu.VMEM((RPW,), jnp.int32),
            pltpu.VMEM((RPW, RPW), jnp.float32),
            pltpu.VMEM((L,), jnp.float32),
            pltpu.SemaphoreType.DMA,
            pltpu.SemaphoreType.DMA,
            pltpu.SemaphoreType.DMA,
            pltpu.SemaphoreType.DMA,
            pltpu.SemaphoreType.DMA,
            pltpu.SemaphoreType.DMA,
        ],
    )
    def k(inp_hbm, tgt_hbm, out_hbm, tgt_v, val_v, stage_v, st0, st1, g0, g1, g2, g3):
        wid = lax.axis_index("s") * NC + lax.axis_index("c")
        base = wid * RPW
        half = RPW // 2
        quar = RPW // 4
        tcs = [
            pltpu.make_async_copy(
                tgt_hbm.at[pl.ds(base + h * half, half)],
                tgt_v.at[pl.ds(h * half, half)],
                s,
            )
            for h, s in enumerate([st0, st1])
        ]
        tcs[0].start()
        tcs[1].start()
        blk = pl.ds(pl.multiple_of(base, 128), RPW)
        gsem = [g0, g1, g2, g3]
        cps = [
            pltpu.make_async_copy(
                inp_hbm.at[tgt_v.at[pl.ds(q * quar, quar)], blk],
                val_v.at[pl.ds(q * quar, quar)],
                gsem[q],
            )
            for q in range(4)
        ]
        tcs[0].wait()
        cps[0].start()
        cps[1].start()
        tcs[1].wait()
        cps[2].start()
        cps[3].start()
        lanes = lax.iota(jnp.int32, L)

        def grp_body(g, acc):
            tch = tgt_v[pl.ds(g * L, L)]
            grp = jnp.zeros((L,), jnp.float32)
            for j in range(L):
                chunk = val_v[g * L + j, pl.ds(g * L, L)]
                grp = grp + jnp.where(lanes == j, chunk, 0.0)
            return acc + jnp.where(tch > 0, grp, 0.0)

        acc = jnp.zeros((L,), jnp.float32)
        for q in range(4):
            cps[q].wait()
            acc = plsc.parallel_loop(q * 2, q * 2 + 2, carry=acc)(grp_body)
        stage_v[...] = acc
        pltpu.sync_copy(stage_v, out_hbm.at[pl.ds(wid * L, L)])

    return k(inp_t, tgt_flat)


def _reduce_body(p_ref, o_ref):
    o_ref[...] = -jnp.sum(p_ref[...]).reshape(1, 1)


def kernel(input, target):
    tgt = target.reshape(-1).astype(jnp.int32)
    partials = _sc_gather_partials(input.T, tgt)
    out = pl.pallas_call(
        _reduce_body,
        out_shape=jax.ShapeDtypeStruct((1, 1), jnp.float32),
    )(partials.reshape(4, 128))
    return out[0, 0]
